# Initial kernel scaffold; baseline (speedup 1.0000x reference)
import functools

import jax
import jax.numpy as jnp
from jax import lax
from jax.experimental import pallas as pl
from jax.experimental.pallas import tpu as pltpu
from jax.experimental.pallas import tpu_sc as plsc

_N, _E, _H, _T = 10000, 320000, 128, 4
_NTILES = 32
_EPW = _E // _NTILES
_C = 400
_NCH = _EPW // _C
_RPT = _N // 16


def _pp_body(nf_ref, wa2_ref, ba2_ref, pp_ref):
    h = nf_ref[:, 0, :]
    pp = jnp.dot(h, wa2_ref[...], preferred_element_type=jnp.float32)
    pp_ref[0] = pp + ba2_ref[...]


def _pp_call(nf, wa2, ba2):
    return pl.pallas_call(
        _pp_body,
        grid=(_T,),
        in_specs=[
            pl.BlockSpec((_N, 1, _H), lambda t: (0, t, 0)),
            pl.BlockSpec((_H, 2), lambda t: (0, 0)),
            pl.BlockSpec((1, 2), lambda t: (0, 0)),
        ],
        out_specs=pl.BlockSpec((1, _N, 2), lambda t: (t, 0, 0)),
        out_shape=jax.ShapeDtypeStruct((_T, _N, 2), jnp.float32),
    )(nf, wa2, ba2)


def _make_sc_kernel():
    mesh = plsc.VectorSubcoreMesh(core_axis_name="c", subcore_axis_name="s")

    @functools.partial(
        pl.kernel,
        mesh=mesh,
        out_type=(
            jax.ShapeDtypeStruct((2, _T, _N, _H), jnp.float32),
            jax.ShapeDtypeStruct((2, _T, _N, 1), jnp.float32),
        ),
        scratch_types=[
            pltpu.VMEM((_N, 2), jnp.float32),
            pltpu.VMEM((_C,), jnp.int32),
            pltpu.VMEM((_C,), jnp.int32),
            pltpu.VMEM((_C,), jnp.int32),
            pltpu.VMEM((_C, 1), jnp.float32),
            pltpu.VMEM((_C, _H), jnp.float32),
            pltpu.VMEM_SHARED((_N, _H), jnp.float32),
            pltpu.VMEM_SHARED((_N, 1), jnp.float32),
            pltpu.SemaphoreType.DMA,
        ],
    )
    def sck(nf_flat, esrc, edst, pp, zrows, zden, aggout, denout,
            ppb, srcb, dstb, idxb, exb, rows, agg_sp, den_sp, gsem):
        cid = lax.axis_index("c")
        sid = lax.axis_index("s")
        eb = (cid * 16 + sid) * _EPW
        rb = sid * _RPT
        i16 = lax.iota(jnp.int32, 16)
        z16 = jnp.zeros((16,), jnp.int32)
        o16 = jnp.ones((16,), jnp.int32)

        def t_body(t, carry):
            pltpu.sync_copy(pp.at[t], ppb)
            pltpu.sync_copy(zrows.at[pl.ds(rb, _RPT)],
                            agg_sp.at[pl.ds(rb, _RPT)])

            @pl.when(sid == 0)
            def _zero_den():
                pltpu.sync_copy(zden, den_sp)

            plsc.subcore_barrier()

            def c_body(c, carry2):
                e0 = eb + c * _C
                pltpu.sync_copy(esrc.at[pl.ds(e0, _C)], srcb)
                pltpu.sync_copy(edst.at[pl.ds(e0, _C)], dstb)

                def g_body(g, carry3):
                    jv = g * 16 + i16
                    sv = plsc.load_gather(srcb, [jv])
                    dv = plsc.load_gather(dstb, [jv])
                    psv = plsc.load_gather(ppb, [sv, z16])
                    pdv = plsc.load_gather(ppb, [dv, o16])
                    zz = psv + pdv
                    ll = jnp.where(zz >= 0, zz, zz * jnp.float32(0.2))
                    exv = jnp.exp(ll)
                    plsc.store_scatter(exb, [jv, z16], exv)
                    plsc.store_scatter(idxb, [jv], sv * _T + t)
                    return carry3

                lax.fori_loop(0, _C // 16, g_body, 0)
                pltpu.async_copy(nf_flat.at[idxb], rows, gsem).wait()

                def s_body(g, carry3):
                    jv = g * 16 + i16
                    exv = plsc.load_gather(exb, [jv, z16])
                    for kk in range(_H):
                        cv = jnp.full((16,), kk, jnp.int32)
                        v = plsc.load_gather(rows, [jv, cv])
                        plsc.store_scatter(rows, [jv, cv], v * exv)
                    return carry3

                lax.fori_loop(0, _C // 16, s_body, 0)
                pltpu.sync_copy(exb, den_sp.at[dstb], add=True)
                pltpu.sync_copy(rows, agg_sp.at[dstb], add=True)
                return carry2

            lax.fori_loop(0, _NCH, c_body, 0)
            plsc.subcore_barrier()
            pltpu.sync_copy(agg_sp.at[pl.ds(rb, _RPT)],
                            aggout.at[cid, t, pl.ds(rb, _RPT)])

            @pl.when(sid == 0)
            def _out_den():
                pltpu.sync_copy(den_sp, denout.at[cid, t])

            plsc.subcore_barrier()
            return carry

        lax.fori_loop(0, _T, t_body, 0)

    return sck


_sc_kernel = _make_sc_kernel()


def _upd_body(nf_ref, aggp_ref, denp_ref, wn1_ref, wn2_ref, bn_ref, out_ref):
    h = nf_ref[:, 0, :]
    agg = aggp_ref[0, 0] + aggp_ref[1, 0]
    den = denp_ref[0, 0] + denp_ref[1, 0]
    rows = agg / (den + jnp.float32(1e-9))
    upd = (jnp.dot(h, wn1_ref[...], preferred_element_type=jnp.float32)
           + jnp.dot(rows, wn2_ref[...], preferred_element_type=jnp.float32)
           + bn_ref[...])
    out_ref[:, 0, :] = jnp.maximum(upd, 0.0)


def _upd_call(nf, aggp, denp, wn1, wn2, bn2):
    return pl.pallas_call(
        _upd_body,
        grid=(_T,),
        in_specs=[
            pl.BlockSpec((_N, 1, _H), lambda t: (0, t, 0)),
            pl.BlockSpec((2, 1, _N, _H), lambda t: (0, t, 0, 0)),
            pl.BlockSpec((2, 1, _N, 1), lambda t: (0, t, 0, 0)),
            pl.BlockSpec((_H, _H), lambda t: (0, 0)),
            pl.BlockSpec((_H, _H), lambda t: (0, 0)),
            pl.BlockSpec((1, _H), lambda t: (0, 0)),
        ],
        out_specs=pl.BlockSpec((_N, 1, _H), lambda t: (0, t, 0)),
        out_shape=jax.ShapeDtypeStruct((_N, _T, _H), jnp.float32),
    )(nf, aggp, denp, wn1, wn2, bn2)


def kernel(nf, edge_index, Wa, ba, Wn, bn):
    nf = nf.astype(jnp.float32)
    src = edge_index[0]
    dst = edge_index[1]
    wa2 = jnp.concatenate([Wa[:_H], Wa[_H:]], axis=1)
    ba2 = jnp.concatenate(
        [ba.reshape(1, 1), jnp.zeros((1, 1), jnp.float32)], axis=1)
    pp = _pp_call(nf, wa2, ba2)
    nf_flat = nf.reshape(_N * _T, _H)
    zrows = jnp.zeros((_N, _H), jnp.float32)
    zden = jnp.zeros((_N, 1), jnp.float32)
    aggp, denp = _sc_kernel(nf_flat, src, dst, pp, zrows, zden)
    out = _upd_call(nf, aggp, denp, Wn[:_H], Wn[_H:], bn.reshape(1, _H))
    return out


# speedup vs baseline: 3.1635x; 3.1635x over previous
import functools

import jax
import jax.numpy as jnp
from jax import lax
from jax.experimental import pallas as pl
from jax.experimental.pallas import tpu as pltpu
from jax.experimental.pallas import tpu_sc as plsc

_N, _E, _H, _T = 10000, 320000, 128, 4
_NTILES = 32
_EPW = _E // _NTILES
_C = 80
_NCH = _EPW // _C
_NP = 10240
_RPT = _NP // 16


def _pp_body(nf_ref, wa2t_ref, ba2t_ref, pp_ref):
    h = nf_ref[0]
    pp = lax.dot_general(wa2t_ref[...], h, (((1,), (1,)), ((), ())),
                         preferred_element_type=jnp.float32)
    pp_ref[0] = pp + ba2t_ref[...]


def _pp_call(nft, wa2t, ba2t):
    return pl.pallas_call(
        _pp_body,
        grid=(_T,),
        in_specs=[
            pl.BlockSpec((1, _N, _H), lambda t: (t, 0, 0)),
            pl.BlockSpec((2, _H), lambda t: (0, 0)),
            pl.BlockSpec((2, 1), lambda t: (0, 0)),
        ],
        out_specs=pl.BlockSpec((1, 2, _N), lambda t: (t, 0, 0)),
        out_shape=jax.ShapeDtypeStruct((_T, 2, _N), jnp.float32),
    )(nft, wa2t, ba2t)


def _make_sc_kernel():
    mesh = plsc.VectorSubcoreMesh(core_axis_name="c", subcore_axis_name="s")

    @functools.partial(
        pl.kernel,
        mesh=mesh,
        compiler_params=pltpu.CompilerParams(needs_layout_passes=False),
        out_type=(
            jax.ShapeDtypeStruct((2, _T, _N, _H), jnp.float32),
            jax.ShapeDtypeStruct((2, _T, _NP), jnp.float32),
        ),
        scratch_types=[
            pltpu.VMEM((_N,), jnp.float32),
            pltpu.VMEM((_N,), jnp.float32),
            pltpu.VMEM((_C,), jnp.int32),
            pltpu.VMEM((_C,), jnp.int32),
            pltpu.VMEM((_C,), jnp.int32),
            pltpu.VMEM((_C,), jnp.float32),
            pltpu.VMEM((_C, _H), jnp.float32),
            pltpu.VMEM_SHARED((_NP, _H), jnp.float32),
            pltpu.VMEM_SHARED((_NP,), jnp.float32),
            pltpu.SemaphoreType.DMA,
        ],
    )
    def sck(nf_flat, esrc, edst, pp, zrows, zden, aggout, denout,
            psb, pdb, srcb, dstb, idxb, exb, rows, agg_sp, den_sp, gsem):
        cid = lax.axis_index("c")
        sid = lax.axis_index("s")
        eb = (cid * 16 + sid) * _EPW
        rb = sid * _RPT
        i16 = lax.iota(jnp.int32, 16)

        def t_body(t, carry):
            pltpu.sync_copy(pp.at[t, 0], psb)
            pltpu.sync_copy(pp.at[t, 1], pdb)
            pltpu.sync_copy(zrows.at[pl.ds(rb, _RPT)],
                            agg_sp.at[pl.ds(rb, _RPT)])

            @pl.when(sid == 0)
            def _zero_den():
                pltpu.sync_copy(zden, den_sp)

            plsc.subcore_barrier()

            def c_body(c, carry2):
                e0 = eb + c * _C
                pltpu.sync_copy(esrc.at[pl.ds(e0, _C)], srcb)
                pltpu.sync_copy(edst.at[pl.ds(e0, _C)], dstb)

                def g_body(g, carry3):
                    jv = g * 16 + i16
                    sv = plsc.load_gather(srcb, [jv])
                    dv = plsc.load_gather(dstb, [jv])
                    psv = plsc.load_gather(psb, [sv])
                    pdv = plsc.load_gather(pdb, [dv])
                    zz = psv + pdv
                    ll = jnp.where(zz >= 0, zz, zz * jnp.float32(0.2))
                    exv = jnp.exp(ll)
                    plsc.store_scatter(exb, [jv], exv)
                    plsc.store_scatter(idxb, [jv], sv + t * _N)
                    return carry3

                lax.fori_loop(0, _C // 16, g_body, 0)
                pltpu.async_copy(nf_flat.at[idxb], rows, gsem).wait()

                def s_body(g, carry3):
                    jv = g * 16 + i16
                    exv = plsc.load_gather(exb, [jv])
                    for kk in range(_H):
                        cv = jnp.full((16,), kk, jnp.int32)
                        v = plsc.load_gather(rows, [jv, cv])
                        plsc.store_scatter(rows, [jv, cv], v * exv)
                    return carry3

                lax.fori_loop(0, _C // 16, s_body, 0)
                pltpu.sync_copy(exb, den_sp.at[dstb], add=True)
                pltpu.sync_copy(rows, agg_sp.at[dstb], add=True)
                return carry2

            lax.fori_loop(0, _NCH, c_body, 0)
            plsc.subcore_barrier()

            @pl.when(sid < 15)
            def _out_agg():
                pltpu.sync_copy(agg_sp.at[pl.ds(rb, _RPT)],
                                aggout.at[cid, t, pl.ds(rb, _RPT)])

            @pl.when(sid == 15)
            def _out_agg_tail():
                pltpu.sync_copy(agg_sp.at[pl.ds(15 * _RPT, _N - 15 * _RPT)],
                                aggout.at[cid, t, pl.ds(15 * _RPT, _N - 15 * _RPT)])

            @pl.when(sid == 0)
            def _out_den():
                pltpu.sync_copy(den_sp, denout.at[cid, t])

            plsc.subcore_barrier()
            return carry

        lax.fori_loop(0, _T, t_body, 0)

    return sck


_sc_kernel = _make_sc_kernel()


def _upd_body(nf_ref, aggp_ref, denp_ref, wn1_ref, wn2_ref, bn_ref, out_ref):
    h = nf_ref[0]
    agg = aggp_ref[0, 0] + aggp_ref[1, 0]
    den = denp_ref[0, 0] + denp_ref[1, 0]
    rows = agg / (den + jnp.float32(1e-9))
    upd = (jnp.dot(h, wn1_ref[...], preferred_element_type=jnp.float32)
           + jnp.dot(rows, wn2_ref[...], preferred_element_type=jnp.float32)
           + bn_ref[...])
    out_ref[0] = jnp.maximum(upd, 0.0)


_NB = 2000


def _upd_call(nft, aggp, denp, wn1, wn2, bn2):
    return pl.pallas_call(
        _upd_body,
        grid=(_T, _N // _NB),
        in_specs=[
            pl.BlockSpec((1, _NB, _H), lambda t, b: (t, b, 0)),
            pl.BlockSpec((2, 1, _NB, _H), lambda t, b: (0, t, b, 0)),
            pl.BlockSpec((2, 1, _NB, 1), lambda t, b: (0, t, b, 0)),
            pl.BlockSpec((_H, _H), lambda t, b: (0, 0)),
            pl.BlockSpec((_H, _H), lambda t, b: (0, 0)),
            pl.BlockSpec((1, _H), lambda t, b: (0, 0)),
        ],
        out_specs=pl.BlockSpec((1, _NB, _H), lambda t, b: (t, b, 0)),
        out_shape=jax.ShapeDtypeStruct((_T, _N, _H), jnp.float32),
    )(nft, aggp, denp, wn1, wn2, bn2)


def kernel(nf, edge_index, Wa, ba, Wn, bn):
    nf = nf.astype(jnp.float32)
    src = edge_index[0]
    dst = edge_index[1]
    wa2t = jnp.concatenate([Wa[:_H], Wa[_H:]], axis=1).T
    ba2t = jnp.concatenate(
        [ba.reshape(1, 1), jnp.zeros((1, 1), jnp.float32)], axis=0)
    nft = jnp.transpose(nf, (1, 0, 2))
    pp = _pp_call(nft, wa2t, ba2t)
    nf_flat = nft.reshape(_T * _N, _H)
    zrows = jnp.zeros((_NP, _H), jnp.float32)
    zden = jnp.zeros((_NP,), jnp.float32)
    aggp, denp = _sc_kernel(nf_flat, src, dst, pp, zrows, zden)
    denp = denp[:, :, :_N].reshape(2, _T, _N, 1)
    out = _upd_call(nft, aggp, denp, Wn[:_H], Wn[_H:], bn.reshape(1, _H))
    return jnp.transpose(out, (1, 0, 2))


# speedup vs baseline: 3.6263x; 1.1463x over previous
import functools

import jax
import jax.numpy as jnp
from jax import lax
from jax.experimental import pallas as pl
from jax.experimental.pallas import tpu as pltpu
from jax.experimental.pallas import tpu_sc as plsc

_N, _E, _H, _T = 10000, 320000, 128, 4
_NTILES = 32
_EPW = _E // _NTILES
_SC = 96
_NSUB = _EPW // _SC
_TCNT = _EPW - _NSUB * _SC
_NP = 10112
_RPT = _NP // 16


def _pp_body(nf_ref, wa2t_ref, ba2t_ref, pp_ref):
    h = nf_ref[0]
    pp = lax.dot_general(wa2t_ref[...], h, (((1,), (1,)), ((), ())),
                         preferred_element_type=jnp.float32)
    pp_ref[0] = pp + ba2t_ref[...]


def _pp_call(nft, wa2t, ba2t):
    return pl.pallas_call(
        _pp_body,
        grid=(_T,),
        in_specs=[
            pl.BlockSpec((1, _N, _H), lambda t: (t, 0, 0)),
            pl.BlockSpec((2, _H), lambda t: (0, 0)),
            pl.BlockSpec((2, 1), lambda t: (0, 0)),
        ],
        out_specs=pl.BlockSpec((1, 2, _N), lambda t: (t, 0, 0)),
        out_shape=jax.ShapeDtypeStruct((_T, 2, _N), jnp.float32),
    )(nft, wa2t, ba2t)


def _make_sc_kernel():
    mesh = plsc.VectorSubcoreMesh(core_axis_name="c", subcore_axis_name="s")

    @functools.partial(
        pl.kernel,
        mesh=mesh,
        compiler_params=pltpu.CompilerParams(needs_layout_passes=False),
        out_type=(
            jax.ShapeDtypeStruct((2, _T, _N, _H), jnp.float32),
            jax.ShapeDtypeStruct((2, _T, _NP), jnp.float32),
        ),
        scratch_types=[
            pltpu.VMEM((_N,), jnp.float32),
            pltpu.VMEM((_N,), jnp.float32),
            pltpu.VMEM((_SC,), jnp.int32),
            pltpu.VMEM((_SC,), jnp.int32),
            pltpu.VMEM((_SC,), jnp.int32),
            pltpu.VMEM((_SC,), jnp.int32),
            pltpu.VMEM((_SC,), jnp.int32),
            pltpu.VMEM((_SC,), jnp.int32),
            pltpu.VMEM((_SC,), jnp.int32),
            pltpu.VMEM((_SC,), jnp.int32),
            pltpu.VMEM((_SC,), jnp.float32),
            pltpu.VMEM((_SC,), jnp.float32),
            pltpu.VMEM((_SC, _H), jnp.float32),
            pltpu.VMEM((_SC, _H), jnp.float32),
            pltpu.VMEM((_TCNT,), jnp.int32),
            pltpu.VMEM((_TCNT,), jnp.int32),
            pltpu.VMEM((_TCNT,), jnp.int32),
            pltpu.VMEM((_TCNT,), jnp.float32),
            pltpu.VMEM((_TCNT, _H), jnp.float32),
            pltpu.VMEM_SHARED((_NP, _H), jnp.float32),
            pltpu.VMEM_SHARED((_NP,), jnp.float32),
            pltpu.SemaphoreType.DMA,
            pltpu.SemaphoreType.DMA,
            pltpu.SemaphoreType.DMA,
            pltpu.SemaphoreType.DMA,
            pltpu.SemaphoreType.DMA,
            pltpu.SemaphoreType.DMA,
            pltpu.SemaphoreType.DMA,
            pltpu.SemaphoreType.DMA,
            pltpu.SemaphoreType.DMA,
        ],
    )
    def sck(nf_flat, esrc, edst, pp, zrows, zden, aggout, denout,
            psb, pdb,
            srcb0, srcb1, dstb0, dstb1, idxb0, idxb1, dsc0, dsc1,
            exb0, exb1, rows0, rows1,
            srct, dstt, idxt, ext, rowst,
            agg_sp, den_sp,
            lsem0, lsem1, gsem0, gsem1, rsem0, rsem1, esem0, esem1, tsem):
        cid = lax.axis_index("c")
        sid = lax.axis_index("s")
        eb = (cid * 16 + sid) * _EPW
        rb = sid * _RPT
        i16 = lax.iota(jnp.int32, 16)
        srcb = (srcb0, srcb1)
        dstb = (dstb0, dstb1)
        idxb = (idxb0, idxb1)
        dsc = (dsc0, dsc1)
        exb = (exb0, exb1)
        rows = (rows0, rows1)
        lsem = (lsem0, lsem1)
        gsem = (gsem0, gsem1)
        rsem = (rsem0, rsem1)
        esem = (esem0, esem1)

        def start_loads(b, k):
            e0 = eb + k * _SC
            pltpu.async_copy(esrc.at[pl.ds(e0, _SC)], srcb[b], lsem[b])
            pltpu.async_copy(edst.at[pl.ds(e0, _SC)], dstb[b], lsem[b])

        def wait_loads(b, k):
            e0 = eb + k * _SC
            pltpu.make_async_copy(esrc.at[pl.ds(e0, _SC)], srcb[b],
                                  lsem[b]).wait()
            pltpu.make_async_copy(edst.at[pl.ds(e0, _SC)], dstb[b],
                                  lsem[b]).wait()

        def ex_idx_phase(b, t):
            tn = t * _N
            for g in range(_SC // 16):
                jv = g * 16 + i16
                sv = plsc.load_gather(srcb[b], [jv])
                dv = plsc.load_gather(dstb[b], [jv])
                psv = plsc.load_gather(psb, [sv])
                pdv = plsc.load_gather(pdb, [dv])
                zz = psv + pdv
                ll = jnp.where(zz >= 0, zz, zz * jnp.float32(0.2))
                exv = jnp.exp(ll)
                plsc.store_scatter(exb[b], [jv], exv)
                plsc.store_scatter(idxb[b], [jv], sv + tn)
                plsc.store_scatter(dsc[b], [jv], dv)

        def scale_phase(b):
            def g_body(g, c):
                jv = g * 16 + i16
                exv = plsc.load_gather(exb[b], [jv])
                for kk in range(_H):
                    cv = jnp.full((16,), kk, jnp.int32)
                    v = plsc.load_gather(rows[b], [jv, cv])
                    plsc.store_scatter(rows[b], [jv, cv], v * exv)
                return c
            lax.fori_loop(0, _SC // 16, g_body, 0)

        def drain_scatters(b):
            pltpu.make_async_copy(rows[b], agg_sp.at[dsc[b]], rsem[b]).wait()
            pltpu.make_async_copy(exb[b], den_sp.at[dsc[b]], esem[b]).wait()

        def scale_and_scatter(b):
            pltpu.make_async_copy(nf_flat.at[idxb[b]], rows[b],
                                  gsem[b]).wait()
            scale_phase(b)
            pltpu.async_copy(rows[b], agg_sp.at[dsc[b]], rsem[b], add=True)
            pltpu.async_copy(exb[b], den_sp.at[dsc[b]], esem[b], add=True)

        def t_body(t, carry):
            pltpu.sync_copy(zrows.at[pl.ds(rb, _RPT)],
                            agg_sp.at[pl.ds(rb, _RPT)])

            @pl.when(sid == 1)
            def _zero_den():
                pltpu.sync_copy(zden, den_sp)

            pltpu.sync_copy(pp.at[t, 0], psb)
            pltpu.sync_copy(pp.at[t, 1], pdb)
            plsc.subcore_barrier()

            start_loads(0, 0)
            start_loads(1, 1)

            def pair_body(j, c2):
                k0 = 2 * j

                @pl.when(j >= 1)
                def _():
                    drain_scatters(0)
                wait_loads(0, k0)
                ex_idx_phase(0, t)

                @pl.when(k0 + 2 <= _NSUB - 1)
                def _():
                    start_loads(0, k0 + 2)
                pltpu.async_copy(nf_flat.at[idxb[0]], rows[0], gsem[0])

                @pl.when(j >= 1)
                def _():
                    scale_and_scatter(1)

                k1 = 2 * j + 1

                @pl.when(j >= 1)
                def _():
                    drain_scatters(1)
                wait_loads(1, k1)
                ex_idx_phase(1, t)

                @pl.when(k1 + 2 <= _NSUB - 1)
                def _():
                    start_loads(1, k1 + 2)
                pltpu.async_copy(nf_flat.at[idxb[1]], rows[1], gsem[1])
                scale_and_scatter(0)
                return c2

            lax.fori_loop(0, _NSUB // 2, pair_body, 0)
            scale_and_scatter(1)
            drain_scatters(0)
            drain_scatters(1)

            et = eb + _NSUB * _SC
            pltpu.sync_copy(esrc.at[pl.ds(et, _TCNT)], srct)
            pltpu.sync_copy(edst.at[pl.ds(et, _TCNT)], dstt)
            sv = plsc.load_gather(srct, [i16])
            dv = plsc.load_gather(dstt, [i16])
            psv = plsc.load_gather(psb, [sv])
            pdv = plsc.load_gather(pdb, [dv])
            zz = psv + pdv
            ll = jnp.where(zz >= 0, zz, zz * jnp.float32(0.2))
            exv = jnp.exp(ll)
            plsc.store_scatter(ext, [i16], exv)
            plsc.store_scatter(idxt, [i16], sv + t * _N)
            pltpu.async_copy(nf_flat.at[idxt], rowst, tsem).wait()
            for kk in range(_H):
                cv = jnp.full((16,), kk, jnp.int32)
                v = plsc.load_gather(rowst, [i16, cv])
                plsc.store_scatter(rowst, [i16, cv], v * exv)
            pltpu.sync_copy(rowst, agg_sp.at[dstt], add=True)
            pltpu.sync_copy(ext, den_sp.at[dstt], add=True)

            plsc.subcore_barrier()

            @pl.when(sid < 15)
            def _out_agg():
                pltpu.sync_copy(agg_sp.at[pl.ds(rb, _RPT)],
                                aggout.at[cid, t, pl.ds(rb, _RPT)])

            @pl.when(sid == 15)
            def _out_agg_tail():
                pltpu.sync_copy(agg_sp.at[pl.ds(15 * _RPT, _N - 15 * _RPT)],
                                aggout.at[cid, t, pl.ds(15 * _RPT,
                                                        _N - 15 * _RPT)])

            @pl.when(sid == 0)
            def _out_den():
                pltpu.sync_copy(den_sp, denout.at[cid, t])

            plsc.subcore_barrier()
            return carry

        lax.fori_loop(0, _T, t_body, 0)

    return sck


_sc_kernel = _make_sc_kernel()


def _upd_body(nf_ref, aggp_ref, denp_ref, wn1_ref, wn2_ref, bn_ref, out_ref):
    h = nf_ref[0]
    agg = aggp_ref[0, 0] + aggp_ref[1, 0]
    den = denp_ref[0, 0] + denp_ref[1, 0]
    rows = agg / (den + jnp.float32(1e-9))
    upd = (jnp.dot(h, wn1_ref[...], preferred_element_type=jnp.float32)
           + jnp.dot(rows, wn2_ref[...], preferred_element_type=jnp.float32)
           + bn_ref[...])
    out_ref[0] = jnp.maximum(upd, 0.0)


_NB = 2000


def _upd_call(nft, aggp, denp, wn1, wn2, bn2):
    return pl.pallas_call(
        _upd_body,
        grid=(_T, _N // _NB),
        in_specs=[
            pl.BlockSpec((1, _NB, _H), lambda t, b: (t, b, 0)),
            pl.BlockSpec((2, 1, _NB, _H), lambda t, b: (0, t, b, 0)),
            pl.BlockSpec((2, 1, _NB, 1), lambda t, b: (0, t, b, 0)),
            pl.BlockSpec((_H, _H), lambda t, b: (0, 0)),
            pl.BlockSpec((_H, _H), lambda t, b: (0, 0)),
            pl.BlockSpec((1, _H), lambda t, b: (0, 0)),
        ],
        out_specs=pl.BlockSpec((1, _NB, _H), lambda t, b: (t, b, 0)),
        out_shape=jax.ShapeDtypeStruct((_T, _N, _H), jnp.float32),
    )(nft, aggp, denp, wn1, wn2, bn2)


def kernel(nf, edge_index, Wa, ba, Wn, bn):
    nf = nf.astype(jnp.float32)
    src = edge_index[0]
    dst = edge_index[1]
    wa2t = jnp.concatenate([Wa[:_H], Wa[_H:]], axis=1).T
    ba2t = jnp.concatenate(
        [ba.reshape(1, 1), jnp.zeros((1, 1), jnp.float32)], axis=0)
    nft = jnp.transpose(nf, (1, 0, 2))
    pp = _pp_call(nft, wa2t, ba2t)
    nf_flat = nft.reshape(_T * _N, _H)
    zrows = jnp.zeros((_NP, _H), jnp.float32)
    zden = jnp.zeros((_NP,), jnp.float32)
    aggp, denp = _sc_kernel(nf_flat, src, dst, pp, zrows, zden)
    denp = denp[:, :, :_N].reshape(2, _T, _N, 1)
    out = _upd_call(nft, aggp, denp, Wn[:_H], Wn[_H:], bn.reshape(1, _H))
    return jnp.transpose(out, (1, 0, 2))
